# baseline (speedup 1.0000x reference)
import functools

import jax
import jax.numpy as jnp
from jax import lax
from jax.experimental import pallas as pl
from jax.experimental.pallas import tpu as pltpu
from jax.experimental.pallas import tpu_sc as plsc

EXP = 8
DM = 1024
DI = 4096
NTOK = 4096
BLK = 512
NB = 16
NP = NB * BLK
DIK = 2048
KI = DI // DIK
NW = 32
TPW = NTOK // NW
VPW = TPW // 16
RB = 32
NRB = TPW // RB


RBLK = 4
RTB = NTOK // RBLK


def _router_body(x_ref, wr1_ref, br1_ref, wr2_ref, br2_ref, idx_ref, hist_ref):
    x = x_ref[...]
    h = jnp.maximum(
        jnp.dot(x, wr1_ref[...], preferred_element_type=jnp.float32)
        + br1_ref[...], 0.0)
    logits = (jnp.dot(h, wr2_ref[...], preferred_element_type=jnp.float32)
              + br2_ref[...])
    am = jnp.argmax(logits, axis=-1).astype(jnp.int32)
    idx_ref[0, 0, :] = am
    oh = (am[:, None] == lax.broadcasted_iota(jnp.int32, (1, 16), 1))
    hist_ref[0, 0, :] = jnp.sum(oh.astype(jnp.int32), axis=0)


def _router(x, Wr1, br1, Wr2, br2):
    idx, hist = pl.pallas_call(
        _router_body,
        grid=(RBLK,),
        in_specs=[
            pl.BlockSpec((RTB, DM), lambda i: (i, 0)),
            pl.BlockSpec((DM, DM), lambda i: (0, 0)),
            pl.BlockSpec((1, DM), lambda i: (0, 0)),
            pl.BlockSpec((DM, EXP), lambda i: (0, 0)),
            pl.BlockSpec((1, EXP), lambda i: (0, 0)),
        ],
        out_specs=[
            pl.BlockSpec((1, 1, RTB), lambda i: (i, 0, 0)),
            pl.BlockSpec((1, 1, 16), lambda i: (i, 0, 0)),
        ],
        out_shape=[
            jax.ShapeDtypeStruct((RBLK, 1, RTB), jnp.int32),
            jax.ShapeDtypeStruct((RBLK, 1, 16), jnp.int32),
        ],
    )(x, Wr1, br1.reshape(1, DM), Wr2, br2.reshape(1, EXP))
    return idx.reshape(NTOK), hist.reshape(RBLK * 16)


def _lane_extract(vec, iota, e):
    return jnp.sum(jnp.where(iota == e, vec, 0))


def _dispatch_body(idx_hbm, hist_hbm, x_hbm, xs_hbm, dest_hbm, meta_hbm,
                   idx_v, hist_v, dest_v, rows0, rows1, rows2, meta_v,
                   rs0, rs1, rs2, ws0, ws1, ws2):
    wid = lax.axis_index("s") * 2 + lax.axis_index("c")
    rows = (rows0, rows1, rows2)
    rsem = (rs0, rs1, rs2)
    wsem = (ws0, ws1, ws2)
    rdesc = [
        pltpu.async_copy(x_hbm.at[pl.ds(wid * TPW + j * RB, RB)],
                         rows[j], rsem[j])
        for j in range(3)
    ]
    pltpu.sync_copy(idx_hbm, idx_v)
    pltpu.sync_copy(hist_hbm, hist_v)
    iota = lax.iota(jnp.int32, 16)

    def hist_step(j, cnt):
        v = idx_v[pl.ds(j * 16, 16)]
        for e in range(EXP):
            c = jnp.sum((v == e).astype(jnp.int32))
            cnt = jnp.where(iota == e, cnt + c, cnt)
        return cnt

    def hadd(b, acc):
        return acc + hist_v[pl.ds(b * 16, 16)]

    zero16 = jnp.zeros((16,), jnp.int32)
    kb = wid // (RTB // TPW)
    pre = lax.fori_loop(0, kb, hadd, zero16)
    total_cnt = lax.fori_loop(0, RBLK, hadd, zero16)
    prefix_cnt = lax.fori_loop(kb * (RTB // 16), wid * VPW, hist_step, pre)

    padded = ((total_cnt + (BLK - 1)) // BLK) * BLK
    incl = plsc.cumsum(padded)
    excl = incl - padded
    nblocks = _lane_extract(incl, iota, EXP - 1) // BLK

    incl_s = [_lane_extract(incl, iota, e) for e in range(EXP)]
    lastexp = jnp.int32(0)
    for e in range(EXP):
        lastexp += ((nblocks - 1) * BLK >= incl_s[e]).astype(jnp.int32)

    def block_expert(bvec):
        acc = jnp.zeros((16,), jnp.int32)
        for e in range(EXP):
            acc += (bvec * BLK >= incl_s[e]).astype(jnp.int32)
        return jnp.minimum(acc, lastexp)

    meta_v[pl.ds(0, 16)] = block_expert(iota)
    hi = block_expert(iota + 16)
    meta_v[pl.ds(16, 16)] = jnp.where(iota == (NB - 16), nblocks, hi)

    @pl.when(wid == 0)
    def _():
        pltpu.sync_copy(meta_v, meta_hbm)

    run = prefix_cnt
    for j in range(VPW):
        v = idx_v[pl.ds((wid * VPW + j) * 16, 16)]
        dest = jnp.zeros((16,), jnp.int32)
        for e in range(EXP):
            m = v == e
            pc = plsc.cumsum(m.astype(jnp.int32))
            base = _lane_extract(excl, iota, e) + _lane_extract(run, iota, e)
            dest = jnp.where(m, base + pc - 1, dest)
            run = jnp.where(iota == e, run + jnp.max(pc), run)
        dest_v[j // 2, pl.ds((j % 2) * 16, 16)] = dest

    pltpu.sync_copy(dest_v, dest_hbm.at[wid])

    wdesc = [None] * NRB
    for j in range(3):
        rdesc[j].wait()
        wdesc[j] = pltpu.async_copy(rows[j], xs_hbm.at[dest_v.at[j]], wsem[j])
        if j == 0:
            wdesc[0].wait()
            r3 = pltpu.async_copy(
                x_hbm.at[pl.ds(wid * TPW + 3 * RB, RB)], rows[0], rsem[0])
    r3.wait()
    w3 = pltpu.async_copy(rows[0], xs_hbm.at[dest_v.at[3]], wsem[0])
    wdesc[1].wait()
    wdesc[2].wait()
    w3.wait()


def _dispatch(idx, hist, x):
    mesh = plsc.VectorSubcoreMesh(core_axis_name="c", subcore_axis_name="s")
    f = pl.kernel(
        _dispatch_body,
        out_type=[
            jax.ShapeDtypeStruct((NP, DM), jnp.float32),
            jax.ShapeDtypeStruct((NW, NRB, RB), jnp.int32),
            jax.ShapeDtypeStruct((32,), jnp.int32),
        ],
        mesh=mesh,
        compiler_params=pltpu.CompilerParams(needs_layout_passes=False),
        scratch_types=[
            pltpu.VMEM((NTOK,), jnp.int32),
            pltpu.VMEM((RBLK * 16,), jnp.int32),
            pltpu.VMEM((NRB, RB), jnp.int32),
            pltpu.VMEM((RB, DM), jnp.float32),
            pltpu.VMEM((RB, DM), jnp.float32),
            pltpu.VMEM((RB, DM), jnp.float32),
            pltpu.VMEM((32,), jnp.int32),
            pltpu.SemaphoreType.DMA,
            pltpu.SemaphoreType.DMA,
            pltpu.SemaphoreType.DMA,
            pltpu.SemaphoreType.DMA,
            pltpu.SemaphoreType.DMA,
            pltpu.SemaphoreType.DMA,
        ],
    )
    return f(idx, hist, x)


def _ffn_body(meta_ref, xs_ref, w1_ref, b1_ref, w2_ref, b2_ref,
              lns_ref, lnb_ref, ys_ref, xn_ref, acc_ref):
    i = pl.program_id(0)
    k = pl.program_id(1)
    valid = i < meta_ref[NB]

    @pl.when(valid & (k == 0))
    def _():
        xv = xs_ref[...]
        mu = jnp.mean(xv, axis=-1, keepdims=True)
        var = jnp.mean((xv - mu) ** 2, axis=-1, keepdims=True)
        xn_ref[...] = ((xv - mu) / jnp.sqrt(var + 1e-5) * lns_ref[0, 0]
                       + lnb_ref[0, 0])
        acc_ref[...] = xv + b2_ref[0, 0]

    @pl.when(valid)
    def _():
        h = jnp.maximum(
            jnp.dot(xn_ref[...], w1_ref[0],
                    preferred_element_type=jnp.float32) + b1_ref[0, 0], 0.0)
        acc_ref[...] += jnp.dot(h, w2_ref[0],
                                preferred_element_type=jnp.float32)

    @pl.when(k == KI - 1)
    def _():
        ys_ref[...] = acc_ref[...]


def _ffn(meta, xs, W1, b1, W2, b2, ln_scale, ln_bias):
    def serp(i, k):
        return jnp.where((i % 2) == 0, k, KI - 1 - k)

    def live(i, m):
        return jnp.minimum(i, m[NB] - 1)

    def chunk(i, k, m):
        return jnp.where(i < m[NB], serp(i, k), serp(m[NB] - 1, KI - 1))

    grid_spec = pltpu.PrefetchScalarGridSpec(
        num_scalar_prefetch=1,
        grid=(NB, KI),
        in_specs=[
            pl.BlockSpec((BLK, DM), lambda i, k, m: (live(i, m), 0)),
            pl.BlockSpec((1, DM, DIK), lambda i, k, m: (m[i], 0, chunk(i, k, m))),
            pl.BlockSpec((1, 1, DIK), lambda i, k, m: (m[i], 0, chunk(i, k, m))),
            pl.BlockSpec((1, DIK, DM), lambda i, k, m: (m[i], chunk(i, k, m), 0)),
            pl.BlockSpec((1, 1, DM), lambda i, k, m: (m[i], 0, 0)),
            pl.BlockSpec((1, 1, DM), lambda i, k, m: (m[i], 0, 0)),
            pl.BlockSpec((1, 1, DM), lambda i, k, m: (m[i], 0, 0)),
        ],
        out_specs=pl.BlockSpec((BLK, DM), lambda i, k, m: (live(i, m), 0)),
        scratch_shapes=[
            pltpu.VMEM((BLK, DM), jnp.float32),
            pltpu.VMEM((BLK, DM), jnp.float32),
        ],
    )
    return pl.pallas_call(
        _ffn_body,
        grid_spec=grid_spec,
        out_shape=jax.ShapeDtypeStruct((NP, DM), jnp.float32),
    )(meta, xs, W1, b1.reshape(EXP, 1, DI), W2, b2.reshape(EXP, 1, DM),
      ln_scale.reshape(EXP, 1, DM), ln_bias.reshape(EXP, 1, DM))


def _combine_body(ys_hbm, dest_hbm, out_hbm, dest_v, rows0, rows1, rows2,
                  rs0, rs1, rs2, ws0, ws1, ws2):
    wid = lax.axis_index("s") * 2 + lax.axis_index("c")
    rows = (rows0, rows1, rows2)
    rsem = (rs0, rs1, rs2)
    wsem = (ws0, ws1, ws2)
    pltpu.sync_copy(dest_hbm.at[wid], dest_v)
    g = [
        pltpu.async_copy(ys_hbm.at[dest_v.at[j]], rows[j], rsem[j])
        for j in range(3)
    ]
    s = [None] * NRB
    for j in range(3):
        g[j].wait()
        s[j] = pltpu.async_copy(
            rows[j], out_hbm.at[pl.ds(wid * TPW + j * RB, RB)], wsem[j])
        if j == 0:
            s[0].wait()
            g3 = pltpu.async_copy(ys_hbm.at[dest_v.at[3]], rows[0], rsem[0])
    g3.wait()
    s3 = pltpu.async_copy(
        rows[0], out_hbm.at[pl.ds(wid * TPW + 3 * RB, RB)], ws0)
    s[1].wait()
    s[2].wait()
    s3.wait()


def _combine(ys, dest):
    mesh = plsc.VectorSubcoreMesh(core_axis_name="c", subcore_axis_name="s")
    f = pl.kernel(
        _combine_body,
        out_type=jax.ShapeDtypeStruct((NTOK, DM), jnp.float32),
        mesh=mesh,
        compiler_params=pltpu.CompilerParams(needs_layout_passes=False),
        scratch_types=[
            pltpu.VMEM((NRB, RB), jnp.int32),
            pltpu.VMEM((RB, DM), jnp.float32),
            pltpu.VMEM((RB, DM), jnp.float32),
            pltpu.VMEM((RB, DM), jnp.float32),
            pltpu.SemaphoreType.DMA,
            pltpu.SemaphoreType.DMA,
            pltpu.SemaphoreType.DMA,
            pltpu.SemaphoreType.DMA,
            pltpu.SemaphoreType.DMA,
            pltpu.SemaphoreType.DMA,
        ],
    )
    return f(ys, dest)


def kernel(sequences, Wr1, br1, Wr2, br2, ln_scale, ln_bias, W1, b1, W2, b2):
    Bs, Ss, Dm = sequences.shape
    x = sequences.reshape(NTOK, DM)
    idx, hist = _router(x, Wr1, br1, Wr2, br2)
    xs, dest, meta = _dispatch(idx, hist, x)
    ys = _ffn(meta, xs, W1, b1, W2, b2, ln_scale, ln_bias)
    out = _combine(ys, dest)
    return out.reshape(Bs, Ss, Dm)


# speedup vs baseline: 1.0053x; 1.0053x over previous
import functools

import jax
import jax.numpy as jnp
from jax import lax
from jax.experimental import pallas as pl
from jax.experimental.pallas import tpu as pltpu
from jax.experimental.pallas import tpu_sc as plsc

EXP = 8
DM = 1024
DI = 4096
NTOK = 4096
BLK = 512
NB = 15
NP = NB * BLK
DIK = 2048
KI = DI // DIK
NW = 32
TPW = NTOK // NW
VPW = TPW // 16
RB = 32
NRB = TPW // RB


RBLK = 8
RTB = NTOK // RBLK


def _router_body(x_ref, wr1_ref, br1_ref, wr2_ref, br2_ref, idx_ref, hist_ref):
    x = x_ref[...]
    h = jnp.maximum(
        jnp.dot(x, wr1_ref[...], preferred_element_type=jnp.float32)
        + br1_ref[...], 0.0)
    logits = (jnp.dot(h, wr2_ref[...], preferred_element_type=jnp.float32)
              + br2_ref[...])
    am = jnp.argmax(logits, axis=-1).astype(jnp.int32)
    idx_ref[0, 0, :] = am
    oh = (am[:, None] == lax.broadcasted_iota(jnp.int32, (1, 16), 1))
    hist_ref[0, 0, :] = jnp.sum(oh.astype(jnp.int32), axis=0)


def _router(x, Wr1, br1, Wr2, br2):
    idx, hist = pl.pallas_call(
        _router_body,
        grid=(RBLK,),
        in_specs=[
            pl.BlockSpec((RTB, DM), lambda i: (i, 0)),
            pl.BlockSpec((DM, DM), lambda i: (0, 0)),
            pl.BlockSpec((1, DM), lambda i: (0, 0)),
            pl.BlockSpec((DM, EXP), lambda i: (0, 0)),
            pl.BlockSpec((1, EXP), lambda i: (0, 0)),
        ],
        out_specs=[
            pl.BlockSpec((1, 1, RTB), lambda i: (i, 0, 0)),
            pl.BlockSpec((1, 1, 16), lambda i: (i, 0, 0)),
        ],
        out_shape=[
            jax.ShapeDtypeStruct((RBLK, 1, RTB), jnp.int32),
            jax.ShapeDtypeStruct((RBLK, 1, 16), jnp.int32),
        ],
    )(x, Wr1, br1.reshape(1, DM), Wr2, br2.reshape(1, EXP))
    return idx.reshape(NTOK), hist.reshape(RBLK * 16)


def _lane_extract(vec, iota, e):
    return jnp.sum(jnp.where(iota == e, vec, 0))


def _dispatch_body(idx_hbm, hist_hbm, x_hbm, xs_hbm, dest_hbm, meta_hbm,
                   idx_v, hist_v, dest_v, rows0, rows1, rows2, meta_v,
                   rs0, rs1, rs2, ws0, ws1, ws2):
    wid = lax.axis_index("s") * 2 + lax.axis_index("c")
    rows = (rows0, rows1, rows2)
    rsem = (rs0, rs1, rs2)
    wsem = (ws0, ws1, ws2)
    rdesc = [
        pltpu.async_copy(x_hbm.at[pl.ds(wid * TPW + j * RB, RB)],
                         rows[j], rsem[j])
        for j in range(3)
    ]
    pltpu.sync_copy(idx_hbm, idx_v)
    pltpu.sync_copy(hist_hbm, hist_v)
    iota = lax.iota(jnp.int32, 16)

    def hist_step(j, cnt):
        v = idx_v[pl.ds(j * 16, 16)]
        for e in range(EXP):
            c = jnp.sum((v == e).astype(jnp.int32))
            cnt = jnp.where(iota == e, cnt + c, cnt)
        return cnt

    def hadd(b, acc):
        return acc + hist_v[pl.ds(b * 16, 16)]

    zero16 = jnp.zeros((16,), jnp.int32)
    kb = wid // (RTB // TPW)
    pre = lax.fori_loop(0, kb, hadd, zero16)
    total_cnt = lax.fori_loop(0, RBLK, hadd, zero16)
    prefix_cnt = lax.fori_loop(kb * (RTB // 16), wid * VPW, hist_step, pre)

    padded = ((total_cnt + (BLK - 1)) // BLK) * BLK
    incl = plsc.cumsum(padded)
    excl = incl - padded
    nblocks = _lane_extract(incl, iota, EXP - 1) // BLK

    incl_s = [_lane_extract(incl, iota, e) for e in range(EXP)]
    lastexp = jnp.int32(0)
    for e in range(EXP):
        lastexp += ((nblocks - 1) * BLK >= incl_s[e]).astype(jnp.int32)

    def block_expert(bvec):
        acc = jnp.zeros((16,), jnp.int32)
        for e in range(EXP):
            acc += (bvec * BLK >= incl_s[e]).astype(jnp.int32)
        return jnp.minimum(acc, lastexp)

    meta_v[pl.ds(0, 16)] = jnp.where(
        iota == NB, nblocks, block_expert(iota))
    meta_v[pl.ds(16, 16)] = jnp.zeros((16,), jnp.int32)

    @pl.when(wid == 0)
    def _():
        pltpu.sync_copy(meta_v, meta_hbm)

    run = prefix_cnt
    for j in range(VPW):
        v = idx_v[pl.ds((wid * VPW + j) * 16, 16)]
        dest = jnp.zeros((16,), jnp.int32)
        for e in range(EXP):
            m = v == e
            pc = plsc.cumsum(m.astype(jnp.int32))
            base = _lane_extract(excl, iota, e) + _lane_extract(run, iota, e)
            dest = jnp.where(m, base + pc - 1, dest)
            run = jnp.where(iota == e, run + jnp.max(pc), run)
        dest_v[j // 2, pl.ds((j % 2) * 16, 16)] = dest

    pltpu.sync_copy(dest_v, dest_hbm.at[wid])

    wdesc = [None] * NRB
    for j in range(3):
        rdesc[j].wait()
        wdesc[j] = pltpu.async_copy(rows[j], xs_hbm.at[dest_v.at[j]], wsem[j])
        if j == 0:
            wdesc[0].wait()
            r3 = pltpu.async_copy(
                x_hbm.at[pl.ds(wid * TPW + 3 * RB, RB)], rows[0], rsem[0])
    r3.wait()
    w3 = pltpu.async_copy(rows[0], xs_hbm.at[dest_v.at[3]], wsem[0])
    wdesc[1].wait()
    wdesc[2].wait()
    w3.wait()


def _dispatch(idx, hist, x):
    mesh = plsc.VectorSubcoreMesh(core_axis_name="c", subcore_axis_name="s")
    f = pl.kernel(
        _dispatch_body,
        out_type=[
            jax.ShapeDtypeStruct((NP, DM), jnp.float32),
            jax.ShapeDtypeStruct((NW, NRB, RB), jnp.int32),
            jax.ShapeDtypeStruct((32,), jnp.int32),
        ],
        mesh=mesh,
        compiler_params=pltpu.CompilerParams(needs_layout_passes=False),
        scratch_types=[
            pltpu.VMEM((NTOK,), jnp.int32),
            pltpu.VMEM((RBLK * 16,), jnp.int32),
            pltpu.VMEM((NRB, RB), jnp.int32),
            pltpu.VMEM((RB, DM), jnp.float32),
            pltpu.VMEM((RB, DM), jnp.float32),
            pltpu.VMEM((RB, DM), jnp.float32),
            pltpu.VMEM((32,), jnp.int32),
            pltpu.SemaphoreType.DMA,
            pltpu.SemaphoreType.DMA,
            pltpu.SemaphoreType.DMA,
            pltpu.SemaphoreType.DMA,
            pltpu.SemaphoreType.DMA,
            pltpu.SemaphoreType.DMA,
        ],
    )
    return f(idx, hist, x)


def _ffn_body(meta_ref, xs_ref, w1_ref, b1_ref, w2_ref, b2_ref,
              lns_ref, lnb_ref, ys_ref, xn_ref, acc_ref):
    i = pl.program_id(0)
    k = pl.program_id(1)
    valid = i < meta_ref[NB]

    @pl.when(valid & (k == 0))
    def _():
        xv = xs_ref[...]
        mu = jnp.mean(xv, axis=-1, keepdims=True)
        var = jnp.mean((xv - mu) ** 2, axis=-1, keepdims=True)
        xn_ref[...] = ((xv - mu) / jnp.sqrt(var + 1e-5) * lns_ref[0, 0]
                       + lnb_ref[0, 0])
        acc_ref[...] = xv + b2_ref[0, 0]

    @pl.when(valid)
    def _():
        h = jnp.maximum(
            jnp.dot(xn_ref[...], w1_ref[0],
                    preferred_element_type=jnp.float32) + b1_ref[0, 0], 0.0)
        acc_ref[...] += jnp.dot(h, w2_ref[0],
                                preferred_element_type=jnp.float32)

    @pl.when(k == KI - 1)
    def _():
        ys_ref[...] = acc_ref[...]


def _ffn(meta, xs, W1, b1, W2, b2, ln_scale, ln_bias):
    def serp(i, k):
        return jnp.where((i % 2) == 0, k, KI - 1 - k)

    def live(i, m):
        return jnp.minimum(i, m[NB] - 1)

    def chunk(i, k, m):
        return jnp.where(i < m[NB], serp(i, k), serp(m[NB] - 1, KI - 1))

    grid_spec = pltpu.PrefetchScalarGridSpec(
        num_scalar_prefetch=1,
        grid=(NB, KI),
        in_specs=[
            pl.BlockSpec((BLK, DM), lambda i, k, m: (live(i, m), 0)),
            pl.BlockSpec((1, DM, DIK), lambda i, k, m: (m[i], 0, chunk(i, k, m))),
            pl.BlockSpec((1, 1, DIK), lambda i, k, m: (m[i], 0, chunk(i, k, m))),
            pl.BlockSpec((1, DIK, DM), lambda i, k, m: (m[i], chunk(i, k, m), 0)),
            pl.BlockSpec((1, 1, DM), lambda i, k, m: (m[i], 0, 0)),
            pl.BlockSpec((1, 1, DM), lambda i, k, m: (m[i], 0, 0)),
            pl.BlockSpec((1, 1, DM), lambda i, k, m: (m[i], 0, 0)),
        ],
        out_specs=pl.BlockSpec((BLK, DM), lambda i, k, m: (live(i, m), 0)),
        scratch_shapes=[
            pltpu.VMEM((BLK, DM), jnp.float32),
            pltpu.VMEM((BLK, DM), jnp.float32),
        ],
    )
    return pl.pallas_call(
        _ffn_body,
        grid_spec=grid_spec,
        out_shape=jax.ShapeDtypeStruct((NP, DM), jnp.float32),
    )(meta, xs, W1, b1.reshape(EXP, 1, DI), W2, b2.reshape(EXP, 1, DM),
      ln_scale.reshape(EXP, 1, DM), ln_bias.reshape(EXP, 1, DM))


def _combine_body(ys_hbm, dest_hbm, out_hbm, dest_v, rows0, rows1, rows2,
                  rs0, rs1, rs2, ws0, ws1, ws2):
    wid = lax.axis_index("s") * 2 + lax.axis_index("c")
    rows = (rows0, rows1, rows2)
    rsem = (rs0, rs1, rs2)
    wsem = (ws0, ws1, ws2)
    pltpu.sync_copy(dest_hbm.at[wid], dest_v)
    g = [
        pltpu.async_copy(ys_hbm.at[dest_v.at[j]], rows[j], rsem[j])
        for j in range(3)
    ]
    s = [None] * NRB
    for j in range(3):
        g[j].wait()
        s[j] = pltpu.async_copy(
            rows[j], out_hbm.at[pl.ds(wid * TPW + j * RB, RB)], wsem[j])
        if j == 0:
            s[0].wait()
            g3 = pltpu.async_copy(ys_hbm.at[dest_v.at[3]], rows[0], rsem[0])
    g3.wait()
    s3 = pltpu.async_copy(
        rows[0], out_hbm.at[pl.ds(wid * TPW + 3 * RB, RB)], ws0)
    s[1].wait()
    s[2].wait()
    s3.wait()


def _combine(ys, dest):
    mesh = plsc.VectorSubcoreMesh(core_axis_name="c", subcore_axis_name="s")
    f = pl.kernel(
        _combine_body,
        out_type=jax.ShapeDtypeStruct((NTOK, DM), jnp.float32),
        mesh=mesh,
        compiler_params=pltpu.CompilerParams(needs_layout_passes=False),
        scratch_types=[
            pltpu.VMEM((NRB, RB), jnp.int32),
            pltpu.VMEM((RB, DM), jnp.float32),
            pltpu.VMEM((RB, DM), jnp.float32),
            pltpu.VMEM((RB, DM), jnp.float32),
            pltpu.SemaphoreType.DMA,
            pltpu.SemaphoreType.DMA,
            pltpu.SemaphoreType.DMA,
            pltpu.SemaphoreType.DMA,
            pltpu.SemaphoreType.DMA,
            pltpu.SemaphoreType.DMA,
        ],
    )
    return f(ys, dest)


def kernel(sequences, Wr1, br1, Wr2, br2, ln_scale, ln_bias, W1, b1, W2, b2):
    Bs, Ss, Dm = sequences.shape
    x = sequences.reshape(NTOK, DM)
    idx, hist = _router(x, Wr1, br1, Wr2, br2)
    xs, dest, meta = _dispatch(idx, hist, x)
    ys = _ffn(meta, xs, W1, b1, W2, b2, ln_scale, ln_bias)
    out = _combine(ys, dest)
    return out.reshape(Bs, Ss, Dm)
